# Initial kernel scaffold; baseline (speedup 1.0000x reference)
import jax
import jax.numpy as jnp
from jax.experimental import pallas as pl


def kernel(inputs, table):
    raise NotImplementedError("write your pallas kernel here")


# speedup vs baseline: 2.5768x; 2.5768x over previous
import functools

import numpy as np
import jax
import jax.numpy as jnp
from jax import lax
from jax.experimental import pallas as pl
from jax.experimental.pallas import tpu as pltpu
from jax.experimental.pallas import tpu_sc as plsc

_NC, _NS, _L = 2, 16, 16
_NW = _NC * _NS


@functools.lru_cache(maxsize=None)
def _build(N, D, G, K):
    M = G * K
    mesh = plsc.VectorSubcoreMesh(core_axis_name="c", subcore_axis_name="s")

    @functools.partial(
        pl.kernel,
        mesh=mesh,
        out_type=jax.ShapeDtypeStruct((N, D), jnp.float32),
        compiler_params=pltpu.CompilerParams(use_tc_tiling_on_sc=False),
        scratch_types=[
            pltpu.VMEM((G, K), jnp.int32),
            pltpu.VMEM((G, K), jnp.int32),
            pltpu.VMEM((2, K, D), jnp.float32),
            pltpu.SemaphoreType.DMA,
        ],
    )
    def k(table_hbm, in_hbm, pat_hbm, out_hbm, idx_v, pat_v, rows_v, gsem):
        wid = lax.axis_index("s") * _NC + lax.axis_index("c")
        base = wid * M

        pltpu.sync_copy(in_hbm.at[wid], idx_v)
        pltpu.sync_copy(pat_hbm, pat_v)

        def add_body(g, c):
            for j in range(K // _L):
                sl = pl.ds(j * _L, _L)
                idx_v[g, sl] = idx_v[g, sl] + pat_v[g, sl]
            return c

        lax.fori_loop(0, G, add_body, 0)

        pltpu.async_copy(table_hbm.at[idx_v.at[0]], rows_v.at[0], gsem)
        pltpu.async_copy(table_hbm.at[idx_v.at[1]], rows_v.at[1], gsem)

        def main_body(g, c):
            b = lax.rem(g, 2)
            pltpu.make_async_copy(
                table_hbm.at[idx_v.at[g]], rows_v.at[b], gsem
            ).wait()
            pltpu.sync_copy(rows_v.at[b], out_hbm.at[pl.ds(base + g * K, K)])

            @pl.when(g + 2 < G)
            def _():
                pltpu.async_copy(table_hbm.at[idx_v.at[g + 2]], rows_v.at[b], gsem)

            return c

        lax.fori_loop(0, G, main_body, 0)

    return k


def kernel(inputs, table):
    B, S = inputs.shape
    V, D = table.shape
    N = B * S
    K = 128
    M = N // _NW
    G = M // K
    step = V // S
    prefix = (np.arange(S, dtype=np.int64) * step).astype(np.int32)
    pattern = np.tile(prefix, M // S).reshape(G, K)
    k = _build(N, D, G, K)
    out = k(table, inputs.reshape(_NW, G, K), jnp.asarray(pattern))
    return out.reshape(B, S, D)


# speedup vs baseline: 2.6237x; 1.0182x over previous
import functools

import numpy as np
import jax
import jax.numpy as jnp
from jax import lax
from jax.experimental import pallas as pl
from jax.experimental.pallas import tpu as pltpu
from jax.experimental.pallas import tpu_sc as plsc

_NC, _NS, _L = 2, 16, 16
_NW = _NC * _NS

_CH = 1024
_NB = 3


@functools.lru_cache(maxsize=None)
def _build(N, D, G2):
    M = G2 * _CH
    mesh = plsc.VectorSubcoreMesh(core_axis_name="c", subcore_axis_name="s")

    @functools.partial(
        pl.kernel,
        mesh=mesh,
        out_type=jax.ShapeDtypeStruct((_NW * G2, _CH, D), jnp.float32),
        compiler_params=pltpu.CompilerParams(use_tc_tiling_on_sc=False),
        scratch_types=[
            pltpu.VMEM((G2, _CH), jnp.int32),
            pltpu.VMEM((G2, _CH), jnp.int32),
            pltpu.VMEM((_NB, _CH, D), jnp.float32),
            pltpu.SemaphoreType.DMA,
            pltpu.SemaphoreType.DMA,
        ],
    )
    def k(table_hbm, in_hbm, pat_hbm, out_hbm, idx_v, pat_v, rows_v, gsem, wsem):
        wid = lax.axis_index("s") * _NC + lax.axis_index("c")
        obase = wid * G2

        pltpu.sync_copy(in_hbm.at[wid], idx_v)
        pltpu.sync_copy(pat_hbm, pat_v)

        def add_body(g, c):
            for j in range(_CH // _L):
                sl = pl.ds(j * _L, _L)
                idx_v[g, sl] = idx_v[g, sl] + pat_v[g, sl]
            return c

        lax.fori_loop(0, G2, add_body, 0)

        def gather(c, b):
            pltpu.async_copy(
                table_hbm.at[idx_v.at[c]], rows_v.at[b], gsem
            )

        def write(c, b):
            pltpu.async_copy(rows_v.at[b], out_hbm.at[obase + c], wsem)

        gather(0, 0)
        gather(1, 1)
        for c in range(G2):
            if c >= 2:
                pltpu.make_async_copy(
                    rows_v.at[(c - 2) % _NB], out_hbm.at[obase + c - 2], wsem
                ).wait()
            if 2 <= c + 1 < G2:
                gather(c + 1, (c + 1) % _NB)
            pltpu.make_async_copy(
                table_hbm.at[idx_v.at[c]],
                rows_v.at[c % _NB],
                gsem,
            ).wait()
            write(c, c % _NB)
        for c in (G2 - 2, G2 - 1):
            pltpu.make_async_copy(
                rows_v.at[c % _NB], out_hbm.at[obase + c], wsem
            ).wait()

    return k


def kernel(inputs, table):
    B, S = inputs.shape
    V, D = table.shape
    N = B * S
    M = N // _NW
    G2 = M // _CH
    step = V // S
    prefix = (np.arange(S, dtype=np.int64) * step).astype(np.int32)
    pattern = np.tile(prefix, M // S).reshape(G2, _CH)
    k = _build(N, D, G2)
    out = k(table, inputs.reshape(_NW, G2, _CH), jnp.asarray(pattern))
    return out.reshape(B, S, D)


# speedup vs baseline: 2.6255x; 1.0007x over previous
import functools

import numpy as np
import jax
import jax.numpy as jnp
from jax import lax
from jax.experimental import pallas as pl
from jax.experimental.pallas import tpu as pltpu
from jax.experimental.pallas import tpu_sc as plsc

_NC, _NS, _L = 2, 16, 16
_NW = _NC * _NS

_CH = 1024
_NB = 3


@functools.lru_cache(maxsize=None)
def _build(S, B, D, VS):
    N = B * S
    M = N // _NW
    BW = B // _NW
    G2 = M // _CH
    mesh = plsc.VectorSubcoreMesh(core_axis_name="c", subcore_axis_name="s")

    @functools.partial(
        pl.kernel,
        mesh=mesh,
        out_type=jax.ShapeDtypeStruct((N, D), jnp.float32),
        compiler_params=pltpu.CompilerParams(
            use_tc_tiling_on_sc=False, needs_layout_passes=False
        ),
        scratch_types=[
            pltpu.VMEM((S, BW), jnp.int32),
            pltpu.VMEM((M,), jnp.int32),
            pltpu.VMEM((_NB, _CH), jnp.int32),
            pltpu.VMEM((_NB, _CH, D), jnp.float32),
            pltpu.SemaphoreType.DMA,
            pltpu.SemaphoreType.DMA,
        ],
    )
    def k(table_hbm, in_hbm, pat_hbm, out_hbm,
          idx_v, pat_v, fused_v, rows_v, gsem, wsem):
        wid = lax.axis_index("s") * _NC + lax.axis_index("c")
        base = wid * M

        pltpu.sync_copy(in_hbm.at[:, pl.ds(wid * BW, BW)], idx_v)
        pltpu.sync_copy(pat_hbm, pat_v)

        def fuse(c, b):
            def body(j, carry):
                pv = pat_v[pl.ds(c * _CH + j * _L, _L)]
                sv = lax.bitwise_and(pv, 31)
                bv = lax.shift_right_logical(pv, 5)
                fused_v[b, pl.ds(j * _L, _L)] = (
                    plsc.load_gather(idx_v, [sv, bv]) + sv * VS
                )
                return carry

            lax.fori_loop(0, _CH // _L, body, 0)

        def gather(c, b):
            pltpu.async_copy(table_hbm.at[fused_v.at[b]], rows_v.at[b], gsem)

        def wait_gather(c, b):
            pltpu.make_async_copy(
                table_hbm.at[fused_v.at[b]], rows_v.at[b], gsem
            ).wait()

        def write(c, b):
            pltpu.async_copy(
                rows_v.at[b], out_hbm.at[pl.ds(base + c * _CH, _CH)], wsem
            )

        def wait_write(c, b):
            pltpu.make_async_copy(
                rows_v.at[b], out_hbm.at[pl.ds(base + c * _CH, _CH)], wsem
            ).wait()

        fuse(0, 0)
        gather(0, 0)
        fuse(1, 1)
        gather(1, 1)
        for c in range(G2):
            b = c % _NB
            wait_gather(c, b)
            write(c, b)
            if c + 2 < G2:
                nb = (c + 2) % _NB
                if c >= 1:
                    wait_write(c - 1, nb)
                fuse(c + 2, nb)
                gather(c + 2, nb)
        for c in range(max(0, G2 - 3), G2):
            wait_write(c, c % _NB)

    return k


def kernel(inputs, table):
    B, S = inputs.shape
    V, D = table.shape
    VS = V // S
    M = (B * S) // _NW
    q = np.arange(M, dtype=np.int64)
    pat = ((q // S) * 32 + (q % S)).astype(np.int32)
    k = _build(S, B, D, VS)
    out = k(table, inputs.T, jnp.asarray(pat))
    return out.reshape(B, S, D)
